# baseline (speedup 1.0000x reference)
import functools

import jax
import jax.numpy as jnp
from jax import lax
from jax.experimental import pallas as pl
from jax.experimental.pallas import tpu as pltpu
from jax.experimental.pallas import tpu_sc as plsc

NSEG = 4
CELL = 512
BLK = 256
G = 64
G2 = 8
EPW = 2


def _gelu_exact(x):
    return 0.5 * x * (1.0 + lax.erf(x * 0.7071067811865476))


def _make_sc_gather(T, D, e0):
    NITER = CELL // 16
    NCW = T // CELL
    PW = EPW * T
    mesh = plsc.VectorSubcoreMesh(core_axis_name="c", subcore_axis_name="s")

    @functools.partial(
        pl.kernel,
        mesh=mesh,
        out_type=[
            jax.ShapeDtypeStruct((PW, D), jnp.float32),
            jax.ShapeDtypeStruct((PW,), jnp.float32),
        ],
        scratch_types=[
            pltpu.VMEM((CELL,), jnp.float32),
            pltpu.VMEM((CELL,), jnp.float32),
            pltpu.VMEM((CELL,), jnp.int32),
            pltpu.VMEM((CELL,), jnp.float32),
            pltpu.VMEM((G, D), jnp.float32),
            pltpu.VMEM((16,), jnp.int32),
            pltpu.SemaphoreType.DMA,
        ],
        compiler_params=pltpu.CompilerParams(needs_layout_passes=False),
    )
    def sc_gather(fdT_hbm, fcT_hbm, flat_hbm, bases_hbm,
                  xcomp_hbm, fccomp_hbm,
                  fd_v, fc_v, idx_v, fcc_v, rows_v, base_v, sem):
        wid = lax.axis_index("s") * 2 + lax.axis_index("c")
        e = e0 + wid // NCW
        cell = wid % NCW
        tok0 = cell * CELL

        pltpu.sync_copy(bases_hbm.at[wid], base_v)
        lanes = lax.iota(jnp.int32, 16)
        base = base_v[...][0]

        pltpu.sync_copy(fdT_hbm.at[e, pl.ds(tok0, CELL)], fd_v)
        pltpu.sync_copy(fcT_hbm.at[e, pl.ds(tok0, CELL)], fc_v)

        def zbody(i, _):
            idx_v[pl.ds(i * 16, 16)] = jnp.zeros((16,), jnp.int32)
            return 0
        lax.fori_loop(0, NITER, zbody, 0)

        def cbody(i, ptr):
            fd16 = fd_v[pl.ds(i * 16, 16)]
            m = fd16 > 0.0
            ids = lanes + (tok0 + i * 16)
            _, sids, _ = plsc.sort_key_val(lanes, ids, mask=m)
            idx_v[pl.ds(ptr, 16)] = sids
            fc16 = fc_v[pl.ds(i * 16, 16)]
            _, sfc, _ = plsc.sort_key_val(lanes, fc16, mask=m)
            fcc_v[pl.ds(ptr, 16)] = sfc
            return ptr + plsc.all_reduce_population_count(m)[0]
        cnt = lax.fori_loop(0, NITER, cbody, jnp.int32(0))

        nfull = cnt // G

        def gbody(j, _):
            idxs = idx_v.at[pl.ds(j * G, G)]
            pltpu.async_copy(flat_hbm.at[idxs], rows_v, sem).wait()
            dst = pl.multiple_of(base + j * G, G2)
            pltpu.sync_copy(rows_v, xcomp_hbm.at[pl.ds(dst, G)])
            pltpu.sync_copy(fcc_v.at[pl.ds(j * G, G)],
                            fccomp_hbm.at[pl.ds(dst, G)])
            return 0
        lax.fori_loop(0, nfull, gbody, 0)

        rem = cnt - nfull * G
        nres = (rem + (G2 - 1)) // G2
        rows8 = rows_v.at[pl.ds(0, G2)]

        def rbody(j, _):
            off = pl.multiple_of(nfull * G + j * G2, G2)
            idxs = idx_v.at[pl.ds(off, G2)]
            pltpu.async_copy(flat_hbm.at[idxs], rows8, sem).wait()
            dst = pl.multiple_of(base + off, G2)
            pltpu.sync_copy(rows8, xcomp_hbm.at[pl.ds(dst, G2)])
            pltpu.sync_copy(fcc_v.at[pl.ds(off, G2)],
                            fccomp_hbm.at[pl.ds(dst, G2)])
            return 0
        lax.fori_loop(0, nres, rbody, 0)

    return sc_gather


def _grouped_body(be_ref, nb_ref, x_ref, fc_ref, gw_ref, vw_ref, ow_ref,
                  out_ref, *, blk, d, h, hc):
    i = pl.program_id(0)

    @pl.when(i < nb_ref[0])
    def _compute():
        x = x_ref[...]
        acc = jnp.zeros((blk, d), jnp.float32)
        for hi in range(h // hc):
            gw = gw_ref[0, hi * hc:(hi + 1) * hc, :]
            vw = vw_ref[0, hi * hc:(hi + 1) * hc, :]
            ow = ow_ref[0, :, hi * hc:(hi + 1) * hc]
            g = lax.dot_general(x, gw, (((1,), (1,)), ((), ())),
                                preferred_element_type=jnp.float32)
            v = lax.dot_general(x, vw, (((1,), (1,)), ((), ())),
                                preferred_element_type=jnp.float32)
            gv = _gelu_exact(g) * v
            acc = acc + lax.dot_general(gv, ow, (((1,), (1,)), ((), ())),
                                        preferred_element_type=jnp.float32)
        w = fc_ref[0, 0, :].reshape(blk, 1)
        out_ref[...] = acc * w


def _clampmap(i, be, nb):
    j = jnp.maximum(jnp.minimum(i, nb[0] - 1), 0)
    return j


@jax.jit
def kernel(tokens, dispatch_weights, combine_weights, gate_W, value_W, out_W, out_scale):
    B, N, D = tokens.shape
    E = dispatch_weights.shape[-1]
    H = gate_W.shape[1]
    T = B * N
    SEG = T // NSEG
    HC = 512
    NWAVE = E // EPW
    PW = EPW * T
    NBW = PW // BLK
    NCW = T // CELL

    flat = tokens.reshape(T, D)
    fdT = dispatch_weights.reshape(T, E).T
    fcT = combine_weights.reshape(T, E).T
    ow_scaled = out_W * out_scale[:, None, None]

    maskT = fdT > 0
    scnt = maskT.reshape(E, NCW, CELL).sum(-1).astype(jnp.int32)
    subcap = ((scnt + 7) // 8) * 8
    segsub = subcap.reshape(E, NSEG, NCW // NSEG)
    segtot = segsub.sum(-1)
    cap = ((segtot + BLK - 1) // BLK) * BLK

    capw = cap.reshape(NWAVE, EPW * NSEG)
    basew = (jnp.cumsum(capw, axis=1) - capw)
    base_es = basew.reshape(E, NSEG)
    suboff = jnp.cumsum(segsub, axis=-1) - segsub
    subbase = base_es[:, :, None] + suboff
    subbase_flat = subbase.reshape(E, NCW)

    widx = jnp.arange(32)
    bases_tables = []
    for g in range(NWAVE):
        ew = g * EPW + widx // NCW
        cw = widx % NCW
        bw = subbase_flat[ew, cw].astype(jnp.int32)
        bases_tables.append(jnp.broadcast_to(bw[:, None], (32, 16)))

    jidx = jnp.arange(NBW, dtype=jnp.int32)
    block_experts, nbs = [], []
    for g in range(NWAVE):
        caps_flat = capw[g]
        nblk = caps_flat // BLK
        cumnb = jnp.cumsum(nblk)
        nb_g = cumnb[-1]
        reg = jnp.minimum(jnp.searchsorted(cumnb, jidx, side="right"),
                          EPW * NSEG - 1)
        be_g = (g * EPW + reg // NSEG).astype(jnp.int32)
        last_e = be_g[jnp.maximum(nb_g - 1, 0)]
        block_experts.append(jnp.where(jidx < nb_g, be_g, last_e))
        nbs.append(nb_g)

    cum = jnp.cumsum(maskT, axis=1).astype(jnp.int32)
    cellpref = (jnp.cumsum(scnt, axis=1) - scnt)
    cell_of_t = (jnp.arange(T) // CELL).astype(jnp.int32)
    pos = (subbase_flat[:, cell_of_t]
           + (cum - 1 - cellpref[:, cell_of_t]))
    pos = jnp.clip(pos, 0, PW - 1)

    body = functools.partial(_grouped_body, blk=BLK, d=D, h=H, hc=HC)
    grid_spec = pltpu.PrefetchScalarGridSpec(
        num_scalar_prefetch=2,
        grid=(NBW,),
        in_specs=[
            pl.BlockSpec((BLK, D), lambda i, be, nb: (_clampmap(i, be, nb), 0)),
            pl.BlockSpec((1, 1, BLK),
                         lambda i, be, nb: (_clampmap(i, be, nb), 0, 0)),
            pl.BlockSpec((1, H, D), lambda i, be, nb: (be[i], 0, 0)),
            pl.BlockSpec((1, H, D), lambda i, be, nb: (be[i], 0, 0)),
            pl.BlockSpec((1, D, H), lambda i, be, nb: (be[i], 0, 0)),
        ],
        out_specs=pl.BlockSpec((BLK, D),
                               lambda i, be, nb: (_clampmap(i, be, nb), 0)),
    )

    gathered = []
    for g in range(NWAVE):
        sc_gather = _make_sc_gather(T, D, g * EPW)
        gathered.append(sc_gather(fdT, fcT, flat, bases_tables[g]))
    ys = []
    for g in range(NWAVE):
        x_w, fc_w = gathered[g]
        ys.append(pl.pallas_call(
            body,
            grid_spec=grid_spec,
            out_shape=jax.ShapeDtypeStruct((PW, D), jnp.float32),
        )(block_experts[g], nbs[g][None], x_w,
          fc_w.reshape(NBW, 1, BLK), gate_W, value_W, ow_scaled))
    out = jnp.zeros((T, D), jnp.float32)
    for g in range(NWAVE):
        for el in range(EPW):
            e = g * EPW + el
            ye = ys[g][pos[e]]
            out = out + jnp.where(maskT[e][:, None], ye, 0.0)
    return out.reshape(B, N, D)


# speedup vs baseline: 1.5853x; 1.5853x over previous
import functools

import jax
import jax.numpy as jnp
from jax.experimental import pallas as pl


def _gelu_exact(x):
    return 0.5 * x * (1.0 + jax.lax.erf(x * 0.7071067811865476))


def _moe_body(x_ref, fd_ref, fc_ref, gw_ref, vw_ref, ow_ref, out_ref, *, bt, d, h, hc):
    e = pl.program_id(1)

    @pl.when(e == 0)
    def _init():
        out_ref[...] = jnp.zeros_like(out_ref)

    x = x_ref[...]
    acc = jnp.zeros((bt, d), jnp.float32)
    for hi in range(h // hc):
        gw = gw_ref[0, hi * hc:(hi + 1) * hc, :]
        vw = vw_ref[0, hi * hc:(hi + 1) * hc, :]
        ow = ow_ref[0, :, hi * hc:(hi + 1) * hc]
        g = jax.lax.dot_general(x, gw, (((1,), (1,)), ((), ())),
                                preferred_element_type=jnp.float32, precision=jax.lax.Precision.DEFAULT)
        v = jax.lax.dot_general(x, vw, (((1,), (1,)), ((), ())),
                                preferred_element_type=jnp.float32, precision=jax.lax.Precision.DEFAULT)
        gv = _gelu_exact(g) * v
        acc = acc + jax.lax.dot_general(gv, ow, (((1,), (1,)), ((), ())),
                                        preferred_element_type=jnp.float32, precision=jax.lax.Precision.DEFAULT)
    fd = fd_ref[0, 0, :]
    fc = fc_ref[0, 0, :]
    w = jnp.where(fd > 0, fc, 0.0).reshape(bt, 1)
    out_ref[...] += acc * w


@jax.jit
def kernel(tokens, dispatch_weights, combine_weights, gate_W, value_W, out_W, out_scale):
    B, N, D = tokens.shape
    E = dispatch_weights.shape[-1]
    H = gate_W.shape[1]
    T = B * N
    BT = 1024
    HC = 512

    flat = tokens.reshape(T, D)
    fdT = dispatch_weights.reshape(T, E).T.reshape(E, 1, T)
    fcT = combine_weights.reshape(T, E).T.reshape(E, 1, T)
    ow_scaled = out_W * out_scale[:, None, None]

    nt = T // BT
    body = functools.partial(_moe_body, bt=BT, d=D, h=H, hc=HC)
    out = pl.pallas_call(
        body,
        grid=(nt, E),
        in_specs=[
            pl.BlockSpec((BT, D), lambda t, e: (t, 0)),
            pl.BlockSpec((1, 1, BT), lambda t, e: (e, 0, t)),
            pl.BlockSpec((1, 1, BT), lambda t, e: (e, 0, t)),
            pl.BlockSpec((1, H, D), lambda t, e: (e, 0, 0)),
            pl.BlockSpec((1, H, D), lambda t, e: (e, 0, 0)),
            pl.BlockSpec((1, D, H), lambda t, e: (e, 0, 0)),
        ],
        out_specs=pl.BlockSpec((BT, D), lambda t, e: (t, 0)),
        out_shape=jax.ShapeDtypeStruct((T, D), jnp.float32),
    )(flat, fdT, fcT, gate_W, value_W, ow_scaled)
    return out.reshape(B, N, D)


# speedup vs baseline: 1.5880x; 1.0017x over previous
import functools

import jax
import jax.numpy as jnp
from jax.experimental import pallas as pl


def _gelu_exact(x):
    return 0.5 * x * (1.0 + jax.lax.erf(x * 0.7071067811865476))


def _moe_body(x_ref, fd_ref, fc_ref, gw_ref, vw_ref, ow_ref, out_ref, *, bt, d, h, hc):
    e = pl.program_id(1)

    @pl.when(e == 0)
    def _init():
        out_ref[...] = jnp.zeros_like(out_ref)

    x = x_ref[...]
    acc = jnp.zeros((bt, d), jnp.float32)
    for hi in range(h // hc):
        gw = gw_ref[0, hi * hc:(hi + 1) * hc, :]
        vw = vw_ref[0, hi * hc:(hi + 1) * hc, :]
        ow = ow_ref[0, :, hi * hc:(hi + 1) * hc]
        g = jax.lax.dot_general(x, gw, (((1,), (1,)), ((), ())),
                                preferred_element_type=jnp.float32)
        v = jax.lax.dot_general(x, vw, (((1,), (1,)), ((), ())),
                                preferred_element_type=jnp.float32)
        gv = _gelu_exact(g) * v
        acc = acc + jax.lax.dot_general(gv, ow, (((1,), (1,)), ((), ())),
                                        preferred_element_type=jnp.float32)
    fd = fd_ref[0, 0, :]
    fc = fc_ref[0, 0, :]
    w = jnp.where(fd > 0, fc, 0.0).reshape(bt, 1)
    out_ref[...] += acc * w


@jax.jit
def kernel(tokens, dispatch_weights, combine_weights, gate_W, value_W, out_W, out_scale):
    B, N, D = tokens.shape
    E = dispatch_weights.shape[-1]
    H = gate_W.shape[1]
    T = B * N
    BT = 1024
    HC = 512

    flat = tokens.reshape(T, D)
    fdT = dispatch_weights.reshape(T, E).T.reshape(E, 1, T)
    fcT = combine_weights.reshape(T, E).T.reshape(E, 1, T)
    ow_scaled = out_W * out_scale[:, None, None]

    nt = T // BT
    body = functools.partial(_moe_body, bt=BT, d=D, h=H, hc=HC)
    out = pl.pallas_call(
        body,
        grid=(nt, E),
        in_specs=[
            pl.BlockSpec((BT, D), lambda t, e: (t, 0)),
            pl.BlockSpec((1, 1, BT), lambda t, e: (e, 0, t)),
            pl.BlockSpec((1, 1, BT), lambda t, e: (e, 0, t)),
            pl.BlockSpec((1, H, D), lambda t, e: (e, 0, 0)),
            pl.BlockSpec((1, H, D), lambda t, e: (e, 0, 0)),
            pl.BlockSpec((1, D, H), lambda t, e: (e, 0, 0)),
        ],
        out_specs=pl.BlockSpec((BT, D), lambda t, e: (t, 0)),
        out_shape=jax.ShapeDtypeStruct((T, D), jnp.float32),
    )(flat, fdT, fcT, gate_W, value_W, ow_scaled)
    return out.reshape(B, N, D)
